# baseline (speedup 1.0000x reference)
import functools
import math

import jax
import jax.numpy as jnp
from jax.experimental import pallas as pl
from jax.experimental.pallas import tpu as pltpu

NBINS = 9
POOL = 8
H = 512
W = 512
IMGS = 2


def _shift_down(a):
    return jnp.concatenate([jnp.zeros((1, a.shape[1]), a.dtype), a[:-1, :]], axis=0)


def _shift_up(a):
    return jnp.concatenate([a[1:, :], jnp.zeros((1, a.shape[1]), a.dtype)], axis=0)


def _hog_kernel(x_ref, o_ref, s_ref, hs_ref):
    for im in range(IMGS):
        _hog_one(x_ref, o_ref, s_ref, hs_ref, im)


def _hog_one(x_ref, o_ref, s_ref, hs_ref, im):
    s = (x_ref[im, 0] + x_ref[im, 1] + x_ref[im, 2]) * (1.0 / 3.0)
    s = s.astype(jnp.bfloat16).astype(jnp.float32)

    lane = jax.lax.broadcasted_iota(jnp.int32, (H, W), 1)
    first_col = lane == 0
    last_col = lane == (W - 1)

    def shift_right(a):
        return jnp.where(first_col, 0.0, pltpu.roll(a, 1, axis=1))

    def shift_left(a):
        return jnp.where(last_col, 0.0, pltpu.roll(a, W - 1, axis=1))

    sr = shift_right(s)
    sl = shift_left(s)
    zrow = jnp.zeros((1, W), jnp.float32)
    hd = sr - sl
    s_ref[im, 0:1, :] = zrow
    s_ref[im, 1 : H + 1, :] = hd
    s_ref[im, H + 1 : H + 2, :] = zrow
    hs = sr + 2.0 * s + sl
    hs_ref[im, 0:1, :] = zrow
    hs_ref[im, 1 : H + 1, :] = hs
    hs_ref[im, H + 1 : H + 2, :] = zrow

    gx = s_ref[im, 0:H, :] + 2.0 * hd + s_ref[im, 2 : H + 2, :]
    gy = hs_ref[im, 0:H, :] - hs_ref[im, 2 : H + 2, :]

    mag = jnp.sqrt(gx * gx + gy * gy)
    one_minus = 1.0 - mag

    r = gy / gx
    f = jnp.zeros(mag.shape, jnp.float32)
    for k in range(1, NBINS):
        ck = 1.0 / math.tan(k * math.pi / NBINS)
        f = f + (r <= ck).astype(jnp.float32)

    rows_i = jax.lax.broadcasted_iota(jnp.int32, (W, W // POOL), 0)
    cols_i = jax.lax.broadcasted_iota(jnp.int32, (W, W // POOL), 1)
    pmat = jnp.where(rows_i // POOL == cols_i, 1.0 / POOL, 0.0).astype(jnp.float32)
    rows_r = jax.lax.broadcasted_iota(jnp.int32, (H // POOL, H), 0)
    cols_r = jax.lax.broadcasted_iota(jnp.int32, (H // POOL, H), 1)
    rmat = jnp.where(cols_r // POOL == rows_r, 1.0 / POOL, 0.0).astype(jnp.bfloat16)

    f16 = f.astype(jnp.bfloat16)
    mag16 = mag.astype(jnp.bfloat16)
    om16 = one_minus.astype(jnp.bfloat16)
    zero = jnp.zeros_like(mag16)
    for b in range(NBINS):
        vb = jnp.where(
            f16 == float(b),
            mag16,
            jnp.where(f16 == float((b - 1) % NBINS), om16, zero),
        )
        rp = jax.lax.dot_general(
            rmat, vb, (((1,), (0,)), ((), ())),
            preferred_element_type=jnp.float32,
        )
        pooled = jax.lax.dot_general(
            rp, pmat, (((1,), (0,)), ((), ())),
            preferred_element_type=jnp.float32,
        )
        o_ref[im, b, :, :] = pooled


@jax.jit
def kernel(x, weight):
    del weight
    n = x.shape[0]
    return pl.pallas_call(
        _hog_kernel,
        grid=(n // IMGS,),
        in_specs=[pl.BlockSpec((IMGS, 3, H, W), lambda i: (i, 0, 0, 0))],
        out_specs=pl.BlockSpec(
            (IMGS, NBINS, H // POOL, W // POOL), lambda i: (i, 0, 0, 0)
        ),
        out_shape=jax.ShapeDtypeStruct((n, NBINS, H // POOL, W // POOL), jnp.float32),
        scratch_shapes=[
            pltpu.VMEM((IMGS, H + 2, W), jnp.float32),
            pltpu.VMEM((IMGS, H + 2, W), jnp.float32),
        ],
        compiler_params=pltpu.CompilerParams(
            dimension_semantics=("parallel",),
        ),
    )(x)


# speedup vs baseline: 1.0574x; 1.0574x over previous
import functools
import math

import jax
import jax.numpy as jnp
from jax.experimental import pallas as pl
from jax.experimental.pallas import tpu as pltpu

NBINS = 9
POOL = 8
H = 512
W = 512
IMGS = 4


def _shift_down(a):
    return jnp.concatenate([jnp.zeros((1, a.shape[1]), a.dtype), a[:-1, :]], axis=0)


def _shift_up(a):
    return jnp.concatenate([a[1:, :], jnp.zeros((1, a.shape[1]), a.dtype)], axis=0)


def _hog_kernel(x_ref, o_ref, s_ref, hs_ref):
    for im in range(IMGS):
        _hog_one(x_ref, o_ref, s_ref, hs_ref, im)


def _hog_one(x_ref, o_ref, s_ref, hs_ref, im):
    s = (x_ref[im, 0] + x_ref[im, 1] + x_ref[im, 2]) * (1.0 / 3.0)
    s = s.astype(jnp.bfloat16).astype(jnp.float32)

    lane = jax.lax.broadcasted_iota(jnp.int32, (H, W), 1)
    first_col = lane == 0
    last_col = lane == (W - 1)

    def shift_right(a):
        return jnp.where(first_col, 0.0, pltpu.roll(a, 1, axis=1))

    def shift_left(a):
        return jnp.where(last_col, 0.0, pltpu.roll(a, W - 1, axis=1))

    zrow = jnp.zeros((1, W), jnp.float32)
    s_ref[im, 0:1, :] = zrow
    s_ref[im, 1 : H + 1, :] = s
    s_ref[im, H + 1 : H + 2, :] = zrow
    hs = shift_right(s) + 2.0 * s + shift_left(s)
    hs_ref[im, 0:1, :] = zrow
    hs_ref[im, 1 : H + 1, :] = hs
    hs_ref[im, H + 1 : H + 2, :] = zrow

    v = s_ref[im, 0:H, :] + 2.0 * s + s_ref[im, 2 : H + 2, :]
    gx = shift_right(v) - shift_left(v)
    gy = hs_ref[im, 0:H, :] - hs_ref[im, 2 : H + 2, :]

    mag = jnp.sqrt(gx * gx + gy * gy)
    one_minus = 1.0 - mag

    r = gy / gx
    f = jnp.zeros(mag.shape, jnp.float32)
    for k in range(1, NBINS):
        ck = 1.0 / math.tan(k * math.pi / NBINS)
        f = f + (r <= ck).astype(jnp.float32)

    rows_i = jax.lax.broadcasted_iota(jnp.int32, (W, W // POOL), 0)
    cols_i = jax.lax.broadcasted_iota(jnp.int32, (W, W // POOL), 1)
    pmat = jnp.where(rows_i // POOL == cols_i, 1.0 / POOL, 0.0).astype(jnp.float32)
    rows_r = jax.lax.broadcasted_iota(jnp.int32, (H // POOL, H), 0)
    cols_r = jax.lax.broadcasted_iota(jnp.int32, (H // POOL, H), 1)
    rmat = jnp.where(cols_r // POOL == rows_r, 1.0 / POOL, 0.0).astype(jnp.bfloat16)

    f16 = f.astype(jnp.bfloat16)
    mag16 = mag.astype(jnp.bfloat16)
    om16 = one_minus.astype(jnp.bfloat16)
    zero = jnp.zeros_like(mag16)
    for b in range(NBINS):
        vb = jnp.where(
            f16 == float(b),
            mag16,
            jnp.where(f16 == float((b - 1) % NBINS), om16, zero),
        )
        rp = jax.lax.dot_general(
            rmat, vb, (((1,), (0,)), ((), ())),
            preferred_element_type=jnp.float32,
        )
        pooled = jax.lax.dot_general(
            rp, pmat, (((1,), (0,)), ((), ())),
            preferred_element_type=jnp.float32,
        )
        o_ref[im, b, :, :] = pooled


@jax.jit
def kernel(x, weight):
    del weight
    n = x.shape[0]
    return pl.pallas_call(
        _hog_kernel,
        grid=(n // IMGS,),
        in_specs=[pl.BlockSpec((IMGS, 3, H, W), lambda i: (i, 0, 0, 0))],
        out_specs=pl.BlockSpec(
            (IMGS, NBINS, H // POOL, W // POOL), lambda i: (i, 0, 0, 0)
        ),
        out_shape=jax.ShapeDtypeStruct((n, NBINS, H // POOL, W // POOL), jnp.float32),
        scratch_shapes=[
            pltpu.VMEM((IMGS, H + 2, W), jnp.float32),
            pltpu.VMEM((IMGS, H + 2, W), jnp.float32),
        ],
        compiler_params=pltpu.CompilerParams(
            dimension_semantics=("parallel",),
        ),
    )(x)
